# baseline (speedup 1.0000x reference)
import functools

import jax
import jax.numpy as jnp
from jax import lax
from jax.experimental import pallas as pl
from jax.experimental.pallas import tpu as pltpu
from jax.experimental.pallas import tpu_sc as plsc

_N = 10000
_E = 320000
_D = 128
_R = 32
_TD = 768
_B = 4096

_NC, _NS = 2, 16
_NW = _NC * _NS
_CH = 64
_CPW = 160
_EPW = _CPW * _CH
_EP = _NW * _EPW
_CPW0 = 80
_CPW1 = 2 * _CPW - _CPW0
_SLABMAX = 15
_NP = 10240
_NPR = _NP * _R
_RPT = _NP // _NS
_CPT = _NPR // _NS
_ZC = 1280
_SLAB = 16


def _sc_body(x_hbm, src_hbm, dst_hbm, cidx_hbm, s_out, c_out,
             idx_s, idx_d, idx_c, rows0, rows1,
             ones, zcnt, gsem0, gsem1, ssem0, ssem1, csem, s_acc, c_acc):
    c = lax.axis_index("core")
    s = lax.axis_index("subcore")
    w = c * _NS + s
    rows = (rows0, rows1)
    gsem = (gsem0, gsem1)
    ssem = (ssem0, ssem1)

    z16 = jnp.zeros((16,), jnp.float32)
    one16 = jnp.ones((16,), jnp.float32)

    @pl.loop(0, _CH)
    def _(r):
        @pl.loop(0, _D, step=16)
        def _(l):
            rows0.at[r, pl.ds(l, 16)][...] = z16

    @pl.loop(0, _ZC, step=16)
    def _(i):
        zcnt.at[pl.ds(i, 16)][...] = z16

    @pl.loop(0, _CH, step=16)
    def _(i):
        ones.at[pl.ds(i, 16)][...] = one16

    @pl.loop(0, _RPT, step=_CH)
    def _(r):
        pltpu.sync_copy(rows0, s_acc.at[pl.ds(s * _RPT + r, _CH)])

    @pl.loop(0, _CPT, step=_ZC)
    def _(i):
        pltpu.sync_copy(zcnt, c_acc.at[pl.ds(s * _CPT + i, _ZC)])

    plsc.subcore_barrier()

    nslab = lax.select(c == 0, _CPW0 // _SLAB, _CPW1 // _SLAB)
    base0 = lax.select(c == 0, s * _CPW0, _NS * _CPW0 + s * _CPW1)

    @pl.loop(0, _SLABMAX)
    def _(sl):
      @pl.when(sl < nslab)
      def _():
        base = base0 + sl * _SLAB
        pltpu.sync_copy(src_hbm.at[pl.ds(base, _SLAB)], idx_s)
        pltpu.sync_copy(dst_hbm.at[pl.ds(base, _SLAB)], idx_d)
        pltpu.sync_copy(cidx_hbm.at[pl.ds(base, _SLAB)], idx_c)

        pltpu.async_copy(x_hbm.at[idx_s.at[0]], rows[0], gsem[0])

        @pl.loop(0, _SLAB, step=2)
        def _(k):
            for b in range(2):
                j = k + b
                nb = (b + 1) % 2
                pltpu.make_async_copy(x_hbm.at[idx_s.at[j]], rows[b], gsem[b]).wait()

                @pl.when(j >= 1)
                def _():
                    pltpu.make_async_copy(rows[nb], s_acc.at[idx_d.at[j - 1]],
                                          ssem[nb]).wait()

                @pl.when(j + 1 < _SLAB)
                def _():
                    pltpu.async_copy(x_hbm.at[idx_s.at[j + 1]], rows[nb], gsem[nb])

                pltpu.async_copy(rows[b], s_acc.at[idx_d.at[j]], ssem[b], add=True)
                pltpu.async_copy(ones, c_acc.at[idx_c.at[j]], csem, add=True)

        pltpu.make_async_copy(rows[1], s_acc.at[idx_d.at[_SLAB - 1]], ssem[1]).wait()

        @pl.loop(0, _SLAB)
        def _(j):
            pltpu.make_async_copy(ones, c_acc.at[idx_c.at[j]], csem).wait()

    plsc.subcore_barrier()

    pltpu.sync_copy(s_acc.at[pl.ds(s * _RPT, _RPT)],
                    s_out.at[c].at[pl.ds(s * _RPT, _RPT)])
    pltpu.sync_copy(c_acc.at[pl.ds(s * _CPT, _CPT)],
                    c_out.at[c].at[pl.ds(s * _CPT, _CPT)])


@functools.cache
def _sc_scatter():
  return pl.kernel(
    _sc_body,
    out_type=(jax.ShapeDtypeStruct((_NC, _NP, _D), jnp.float32),
              jax.ShapeDtypeStruct((_NC, _NPR), jnp.float32)),
    mesh=plsc.VectorSubcoreMesh(core_axis_name="core", subcore_axis_name="subcore",
                                num_cores=_NC, num_subcores=_NS),
    scratch_types=[
        pltpu.VMEM((_SLAB, _CH), jnp.int32),
        pltpu.VMEM((_SLAB, _CH), jnp.int32),
        pltpu.VMEM((_SLAB, _CH), jnp.int32),
        pltpu.VMEM((_CH, _D), jnp.float32),
        pltpu.VMEM((_CH, _D), jnp.float32),
        pltpu.VMEM((_CH,), jnp.float32),
        pltpu.VMEM((_ZC,), jnp.float32),
        pltpu.SemaphoreType.DMA,
        pltpu.SemaphoreType.DMA,
        pltpu.SemaphoreType.DMA,
        pltpu.SemaphoreType.DMA,
        pltpu.SemaphoreType.DMA,
        pltpu.VMEM_SHARED((_NP, _D), jnp.float32),
        pltpu.VMEM_SHARED((_NPR,), jnp.float32),
    ],
  )


_GBLK = 2000


def _gnn_body(s_ref, c_ref, x_ref, rel_ref, w_ref, wself_ref, o_ref):
    S = s_ref[0] + s_ref[1]
    C = c_ref[0] + c_ref[1]
    deg = jnp.sum(C, axis=1, keepdims=True)
    relsum = jnp.dot(C, rel_ref[...], precision=lax.Precision.HIGHEST)
    agg = jnp.dot(S - relsum, w_ref[...], precision=lax.Precision.HIGHEST)
    agg = agg / jnp.maximum(deg, 1.0)
    self_t = jnp.dot(x_ref[...], wself_ref[...], precision=lax.Precision.HIGHEST)
    o_ref[...] = jnp.tanh(agg + self_t)


def _gnn_finish(s_parts, c_parts, x, rel_emb, W, W_self):
    return pl.pallas_call(
        _gnn_body,
        grid=(_N // _GBLK,),
        in_specs=[
            pl.BlockSpec((_NC, _GBLK, _D), lambda i: (0, i, 0)),
            pl.BlockSpec((_NC, _GBLK, _R), lambda i: (0, i, 0)),
            pl.BlockSpec((_GBLK, _D), lambda i: (i, 0)),
            pl.BlockSpec((_R, _D), lambda i: (0, 0)),
            pl.BlockSpec((_D, _D), lambda i: (0, 0)),
            pl.BlockSpec((_D, _D), lambda i: (0, 0)),
        ],
        out_specs=pl.BlockSpec((_GBLK, _D), lambda i: (i, 0)),
        out_shape=jax.ShapeDtypeStruct((_N, _D), jnp.float32),
    )(s_parts, c_parts, x, rel_emb, W, W_self)


_TBLK = 512


def _mlp_body(ta_ref, tb_ref, w1_ref, b1_ref, w2_ref, b2_ref, oa_ref, ob_ref):
    w1 = w1_ref[...]
    w2 = w2_ref[...]
    b1 = b1_ref[...]
    b2 = b2_ref[...]
    ha = jnp.maximum(jnp.dot(ta_ref[...], w1, precision=lax.Precision.HIGHEST) + b1, 0.0)
    oa_ref[...] = jnp.dot(ha, w2, precision=lax.Precision.HIGHEST) + b2
    hb = jnp.maximum(jnp.dot(tb_ref[...], w1, precision=lax.Precision.HIGHEST) + b1, 0.0)
    ob_ref[...] = jnp.dot(hb, w2, precision=lax.Precision.HIGHEST) + b2


def _text_mlp(ta, tb, W1, b1, W2, b2):
    return pl.pallas_call(
        _mlp_body,
        grid=(_B // _TBLK,),
        in_specs=[
            pl.BlockSpec((_TBLK, _TD), lambda i: (i, 0)),
            pl.BlockSpec((_TBLK, _TD), lambda i: (i, 0)),
            pl.BlockSpec((_TD, 2 * _D), lambda i: (0, 0)),
            pl.BlockSpec((1, 2 * _D), lambda i: (0, 0)),
            pl.BlockSpec((2 * _D, _D), lambda i: (0, 0)),
            pl.BlockSpec((1, _D), lambda i: (0, 0)),
        ],
        out_specs=[
            pl.BlockSpec((_TBLK, _D), lambda i: (i, 0)),
            pl.BlockSpec((_TBLK, _D), lambda i: (i, 0)),
        ],
        out_shape=[
            jax.ShapeDtypeStruct((_B, _D), jnp.float32),
            jax.ShapeDtypeStruct((_B, _D), jnp.float32),
        ],
    )(ta, tb, W1, b1, W2, b2)


def kernel(x, rel_emb, W, W_self, text_features_ab, text_features_bc,
           W1, b1, W2, b2, edge_index, edge_type):
    src = edge_index[0]
    dst = edge_index[1]
    cidx = dst * _R + edge_type
    pad = _EP - _E
    src_p = jnp.concatenate([src, jnp.zeros((pad,), jnp.int32)])
    dst_p = jnp.concatenate([dst, jnp.full((pad,), _NP - 1, jnp.int32)])
    cidx_p = jnp.concatenate([cidx, jnp.full((pad,), _NPR - 1, jnp.int32)])
    src2 = src_p.reshape(_EP // _CH, _CH)
    dst2 = dst_p.reshape(_EP // _CH, _CH)
    cidx2 = cidx_p.reshape(_EP // _CH, _CH)

    s_parts, c_parts = _sc_scatter()(x, src2, dst2, cidx2)
    c_parts = c_parts.reshape(_NC, _NP, _R)

    proj_ab, proj_bc = _text_mlp(text_features_ab, text_features_bc,
                                 W1, b1.reshape(1, -1), W2, b2.reshape(1, -1))
    node = _gnn_finish(s_parts, c_parts, x, rel_emb, W, W_self)
    return (node, proj_ab, proj_bc)


# speedup vs baseline: 1.2570x; 1.2570x over previous
import functools

import jax
import jax.numpy as jnp
from jax import lax
from jax.experimental import pallas as pl
from jax.experimental.pallas import tpu as pltpu
from jax.experimental.pallas import tpu_sc as plsc

_N = 10000
_E = 320000
_D = 128
_R = 32
_TD = 768
_B = 4096

_NC, _NS = 2, 16
_NW = _NC * _NS
_CH = 64
_CPW = 160
_EPW = _CPW * _CH
_EP = _NW * _EPW
_CPW0 = 224
_CPW1 = 2 * _CPW - _CPW0
_SLABMAX = 14
_NP = 10240
_NPR = _NP * _R
_RPT = _NP // _NS
_CPT = _NPR // _NS
_ZC = 1280
_SLAB = 16


def _sc_body(x_hbm, src_hbm, dst_hbm, cidx_hbm, s_out, c_out,
             idx_s, idx_d, idx_c, rows0, rows1,
             ones, zcnt, gsem0, gsem1, ssem0, ssem1, csem, s_acc, c_acc):
    c = lax.axis_index("core")
    s = lax.axis_index("subcore")
    w = c * _NS + s
    rows = (rows0, rows1)
    gsem = (gsem0, gsem1)
    ssem = (ssem0, ssem1)

    z16 = jnp.zeros((16,), jnp.float32)
    one16 = jnp.ones((16,), jnp.float32)

    @pl.loop(0, _CH)
    def _(r):
        @pl.loop(0, _D, step=16)
        def _(l):
            rows0.at[r, pl.ds(l, 16)][...] = z16

    @pl.loop(0, _ZC, step=16)
    def _(i):
        zcnt.at[pl.ds(i, 16)][...] = z16

    @pl.loop(0, _CH, step=16)
    def _(i):
        ones.at[pl.ds(i, 16)][...] = one16

    @pl.loop(0, _RPT, step=_CH)
    def _(r):
        pltpu.sync_copy(rows0, s_acc.at[pl.ds(s * _RPT + r, _CH)])

    @pl.loop(0, _CPT, step=_ZC)
    def _(i):
        pltpu.sync_copy(zcnt, c_acc.at[pl.ds(s * _CPT + i, _ZC)])

    plsc.subcore_barrier()

    nslab = lax.select(c == 0, _CPW0 // _SLAB, _CPW1 // _SLAB)
    base0 = lax.select(c == 0, s * _CPW0, _NS * _CPW0 + s * _CPW1)

    @pl.loop(0, _SLABMAX)
    def _(sl):
      @pl.when(sl < nslab)
      def _():
        base = base0 + sl * _SLAB
        pltpu.sync_copy(src_hbm.at[pl.ds(base, _SLAB)], idx_s)
        pltpu.sync_copy(dst_hbm.at[pl.ds(base, _SLAB)], idx_d)
        pltpu.sync_copy(cidx_hbm.at[pl.ds(base, _SLAB)], idx_c)

        pltpu.async_copy(x_hbm.at[idx_s.at[0]], rows[0], gsem[0])

        @pl.loop(0, _SLAB, step=2)
        def _(k):
            for b in range(2):
                j = k + b
                nb = (b + 1) % 2
                pltpu.make_async_copy(x_hbm.at[idx_s.at[j]], rows[b], gsem[b]).wait()

                @pl.when(j >= 1)
                def _():
                    pltpu.make_async_copy(rows[nb], s_acc.at[idx_d.at[j - 1]],
                                          ssem[nb]).wait()

                @pl.when(j + 1 < _SLAB)
                def _():
                    pltpu.async_copy(x_hbm.at[idx_s.at[j + 1]], rows[nb], gsem[nb])

                pltpu.async_copy(rows[b], s_acc.at[idx_d.at[j]], ssem[b], add=True)
                pltpu.async_copy(ones, c_acc.at[idx_c.at[j]], csem, add=True)

        pltpu.make_async_copy(rows[1], s_acc.at[idx_d.at[_SLAB - 1]], ssem[1]).wait()

        @pl.loop(0, _SLAB)
        def _(j):
            pltpu.make_async_copy(ones, c_acc.at[idx_c.at[j]], csem).wait()

    plsc.subcore_barrier()

    pltpu.sync_copy(s_acc.at[pl.ds(s * _RPT, _RPT)],
                    s_out.at[c].at[pl.ds(s * _RPT, _RPT)])
    pltpu.sync_copy(c_acc.at[pl.ds(s * _CPT, _CPT)],
                    c_out.at[c].at[pl.ds(s * _CPT, _CPT)])


@functools.cache
def _sc_scatter():
  return pl.kernel(
    _sc_body,
    out_type=(jax.ShapeDtypeStruct((_NC, _NP, _D), jnp.float32),
              jax.ShapeDtypeStruct((_NC, _NPR), jnp.float32)),
    mesh=plsc.VectorSubcoreMesh(core_axis_name="core", subcore_axis_name="subcore",
                                num_cores=_NC, num_subcores=_NS),
    scratch_types=[
        pltpu.VMEM((_SLAB, _CH), jnp.int32),
        pltpu.VMEM((_SLAB, _CH), jnp.int32),
        pltpu.VMEM((_SLAB, _CH), jnp.int32),
        pltpu.VMEM((_CH, _D), jnp.float32),
        pltpu.VMEM((_CH, _D), jnp.float32),
        pltpu.VMEM((_CH,), jnp.float32),
        pltpu.VMEM((_ZC,), jnp.float32),
        pltpu.SemaphoreType.DMA,
        pltpu.SemaphoreType.DMA,
        pltpu.SemaphoreType.DMA,
        pltpu.SemaphoreType.DMA,
        pltpu.SemaphoreType.DMA,
        pltpu.VMEM_SHARED((_NP, _D), jnp.float32),
        pltpu.VMEM_SHARED((_NPR,), jnp.float32),
    ],
  )


_GBLK = 2000


def _gnn_body(s_ref, c_ref, x_ref, rel_ref, w_ref, wself_ref, o_ref):
    S = s_ref[0] + s_ref[1]
    C = c_ref[0] + c_ref[1]
    deg = jnp.sum(C, axis=1, keepdims=True)
    relsum = jnp.dot(C, rel_ref[...], precision=lax.Precision.HIGHEST)
    agg = jnp.dot(S - relsum, w_ref[...], precision=lax.Precision.HIGHEST)
    agg = agg / jnp.maximum(deg, 1.0)
    self_t = jnp.dot(x_ref[...], wself_ref[...], precision=lax.Precision.HIGHEST)
    o_ref[...] = jnp.tanh(agg + self_t)


def _gnn_finish(s_parts, c_parts, x, rel_emb, W, W_self):
    return pl.pallas_call(
        _gnn_body,
        grid=(_N // _GBLK,),
        in_specs=[
            pl.BlockSpec((_NC, _GBLK, _D), lambda i: (0, i, 0)),
            pl.BlockSpec((_NC, _GBLK, _R), lambda i: (0, i, 0)),
            pl.BlockSpec((_GBLK, _D), lambda i: (i, 0)),
            pl.BlockSpec((_R, _D), lambda i: (0, 0)),
            pl.BlockSpec((_D, _D), lambda i: (0, 0)),
            pl.BlockSpec((_D, _D), lambda i: (0, 0)),
        ],
        out_specs=pl.BlockSpec((_GBLK, _D), lambda i: (i, 0)),
        out_shape=jax.ShapeDtypeStruct((_N, _D), jnp.float32),
    )(s_parts, c_parts, x, rel_emb, W, W_self)


_TBLK = 512


def _mlp_body(ta_ref, tb_ref, w1_ref, b1_ref, w2_ref, b2_ref, oa_ref, ob_ref):
    w1 = w1_ref[...]
    w2 = w2_ref[...]
    b1 = b1_ref[...]
    b2 = b2_ref[...]
    ha = jnp.maximum(jnp.dot(ta_ref[...], w1, precision=lax.Precision.HIGHEST) + b1, 0.0)
    oa_ref[...] = jnp.dot(ha, w2, precision=lax.Precision.HIGHEST) + b2
    hb = jnp.maximum(jnp.dot(tb_ref[...], w1, precision=lax.Precision.HIGHEST) + b1, 0.0)
    ob_ref[...] = jnp.dot(hb, w2, precision=lax.Precision.HIGHEST) + b2


def _text_mlp(ta, tb, W1, b1, W2, b2):
    return pl.pallas_call(
        _mlp_body,
        grid=(_B // _TBLK,),
        in_specs=[
            pl.BlockSpec((_TBLK, _TD), lambda i: (i, 0)),
            pl.BlockSpec((_TBLK, _TD), lambda i: (i, 0)),
            pl.BlockSpec((_TD, 2 * _D), lambda i: (0, 0)),
            pl.BlockSpec((1, 2 * _D), lambda i: (0, 0)),
            pl.BlockSpec((2 * _D, _D), lambda i: (0, 0)),
            pl.BlockSpec((1, _D), lambda i: (0, 0)),
        ],
        out_specs=[
            pl.BlockSpec((_TBLK, _D), lambda i: (i, 0)),
            pl.BlockSpec((_TBLK, _D), lambda i: (i, 0)),
        ],
        out_shape=[
            jax.ShapeDtypeStruct((_B, _D), jnp.float32),
            jax.ShapeDtypeStruct((_B, _D), jnp.float32),
        ],
    )(ta, tb, W1, b1, W2, b2)


def kernel(x, rel_emb, W, W_self, text_features_ab, text_features_bc,
           W1, b1, W2, b2, edge_index, edge_type):
    src = edge_index[0]
    dst = edge_index[1]
    cidx = dst * _R + edge_type
    pad = _EP - _E
    src_p = jnp.concatenate([src, jnp.zeros((pad,), jnp.int32)])
    dst_p = jnp.concatenate([dst, jnp.full((pad,), _NP - 1, jnp.int32)])
    cidx_p = jnp.concatenate([cidx, jnp.full((pad,), _NPR - 1, jnp.int32)])
    src2 = src_p.reshape(_EP // _CH, _CH)
    dst2 = dst_p.reshape(_EP // _CH, _CH)
    cidx2 = cidx_p.reshape(_EP // _CH, _CH)

    s_parts, c_parts = _sc_scatter()(x, src2, dst2, cidx2)
    c_parts = c_parts.reshape(_NC, _NP, _R)

    proj_ab, proj_bc = _text_mlp(text_features_ab, text_features_bc,
                                 W1, b1.reshape(1, -1), W2, b2.reshape(1, -1))
    node = _gnn_finish(s_parts, c_parts, x, rel_emb, W, W_self)
    return (node, proj_ab, proj_bc)


# speedup vs baseline: 1.5098x; 1.2012x over previous
import functools

import jax
import jax.numpy as jnp
from jax import lax
from jax.experimental import pallas as pl
from jax.experimental.pallas import tpu as pltpu
from jax.experimental.pallas import tpu_sc as plsc

_N = 10000
_E = 320000
_D = 128
_R = 32
_TD = 768
_B = 4096

_NC, _NS = 2, 16
_NW = _NC * _NS
_CH = 64
_CPW = 160
_EPW = _CPW * _CH
_EP = _NW * _EPW
_CPW0 = 272
_CPW1 = 2 * _CPW - _CPW0
_SLABMAX = 17
_NP = 10240
_NPR = _NP * _R
_RPT = _NP // _NS
_CPT = _NPR // _NS
_ZC = 1280
_SLAB = 16


def _sc_body(x_hbm, src_hbm, dst_hbm, cidx_hbm, s_out, c_out,
             idx_s, idx_d, idx_c, rows0, rows1,
             ones, zcnt, gsem0, gsem1, ssem0, ssem1, csem, s_acc, c_acc):
    c = lax.axis_index("core")
    s = lax.axis_index("subcore")
    w = c * _NS + s
    rows = (rows0, rows1)
    gsem = (gsem0, gsem1)
    ssem = (ssem0, ssem1)

    z16 = jnp.zeros((16,), jnp.float32)
    one16 = jnp.ones((16,), jnp.float32)

    @pl.loop(0, _CH)
    def _(r):
        @pl.loop(0, _D, step=16)
        def _(l):
            rows0.at[r, pl.ds(l, 16)][...] = z16

    @pl.loop(0, _ZC, step=16)
    def _(i):
        zcnt.at[pl.ds(i, 16)][...] = z16

    @pl.loop(0, _CH, step=16)
    def _(i):
        ones.at[pl.ds(i, 16)][...] = one16

    @pl.loop(0, _RPT, step=_CH)
    def _(r):
        pltpu.sync_copy(rows0, s_acc.at[pl.ds(s * _RPT + r, _CH)])

    @pl.loop(0, _CPT, step=_ZC)
    def _(i):
        pltpu.sync_copy(zcnt, c_acc.at[pl.ds(s * _CPT + i, _ZC)])

    plsc.subcore_barrier()

    nslab = lax.select(c == 0, _CPW0 // _SLAB, _CPW1 // _SLAB)
    base0 = lax.select(c == 0, s * _CPW0, _NS * _CPW0 + s * _CPW1)

    @pl.loop(0, _SLABMAX)
    def _(sl):
      @pl.when(sl < nslab)
      def _():
        base = base0 + sl * _SLAB
        pltpu.sync_copy(src_hbm.at[pl.ds(base, _SLAB)], idx_s)
        pltpu.sync_copy(dst_hbm.at[pl.ds(base, _SLAB)], idx_d)
        pltpu.sync_copy(cidx_hbm.at[pl.ds(base, _SLAB)], idx_c)

        pltpu.async_copy(x_hbm.at[idx_s.at[0]], rows[0], gsem[0])

        @pl.loop(0, _SLAB, step=2)
        def _(k):
            for b in range(2):
                j = k + b
                nb = (b + 1) % 2
                pltpu.make_async_copy(x_hbm.at[idx_s.at[j]], rows[b], gsem[b]).wait()

                @pl.when(j >= 1)
                def _():
                    pltpu.make_async_copy(rows[nb], s_acc.at[idx_d.at[j - 1]],
                                          ssem[nb]).wait()

                @pl.when(j + 1 < _SLAB)
                def _():
                    pltpu.async_copy(x_hbm.at[idx_s.at[j + 1]], rows[nb], gsem[nb])

                pltpu.async_copy(rows[b], s_acc.at[idx_d.at[j]], ssem[b], add=True)
                pltpu.async_copy(ones, c_acc.at[idx_c.at[j]], csem, add=True)

        pltpu.make_async_copy(rows[1], s_acc.at[idx_d.at[_SLAB - 1]], ssem[1]).wait()

        @pl.loop(0, _SLAB)
        def _(j):
            pltpu.make_async_copy(ones, c_acc.at[idx_c.at[j]], csem).wait()

    plsc.subcore_barrier()

    pltpu.sync_copy(s_acc.at[pl.ds(s * _RPT, _RPT)],
                    s_out.at[c].at[pl.ds(s * _RPT, _RPT)])
    pltpu.sync_copy(c_acc.at[pl.ds(s * _CPT, _CPT)],
                    c_out.at[c].at[pl.ds(s * _CPT, _CPT)])


@functools.cache
def _sc_scatter():
  return pl.kernel(
    _sc_body,
    out_type=(jax.ShapeDtypeStruct((_NC, _NP, _D), jnp.float32),
              jax.ShapeDtypeStruct((_NC, _NPR), jnp.float32)),
    mesh=plsc.VectorSubcoreMesh(core_axis_name="core", subcore_axis_name="subcore",
                                num_cores=_NC, num_subcores=_NS),
    scratch_types=[
        pltpu.VMEM((_SLAB, _CH), jnp.int32),
        pltpu.VMEM((_SLAB, _CH), jnp.int32),
        pltpu.VMEM((_SLAB, _CH), jnp.int32),
        pltpu.VMEM((_CH, _D), jnp.float32),
        pltpu.VMEM((_CH, _D), jnp.float32),
        pltpu.VMEM((_CH,), jnp.float32),
        pltpu.VMEM((_ZC,), jnp.float32),
        pltpu.SemaphoreType.DMA,
        pltpu.SemaphoreType.DMA,
        pltpu.SemaphoreType.DMA,
        pltpu.SemaphoreType.DMA,
        pltpu.SemaphoreType.DMA,
        pltpu.VMEM_SHARED((_NP, _D), jnp.float32),
        pltpu.VMEM_SHARED((_NPR,), jnp.float32),
    ],
  )


_GBLK = 2000


def _gnn_body(s_ref, c_ref, x_ref, rel_ref, w_ref, wself_ref, o_ref):
    S = s_ref[0] + s_ref[1]
    C = c_ref[0] + c_ref[1]
    deg = jnp.sum(C, axis=1, keepdims=True)
    relsum = jnp.dot(C, rel_ref[...], precision=lax.Precision.HIGHEST)
    agg = jnp.dot(S - relsum, w_ref[...], precision=lax.Precision.HIGHEST)
    agg = agg / jnp.maximum(deg, 1.0)
    self_t = jnp.dot(x_ref[...], wself_ref[...], precision=lax.Precision.HIGHEST)
    o_ref[...] = jnp.tanh(agg + self_t)


def _gnn_finish(s_parts, c_parts, x, rel_emb, W, W_self):
    return pl.pallas_call(
        _gnn_body,
        grid=(_N // _GBLK,),
        in_specs=[
            pl.BlockSpec((_NC, _GBLK, _D), lambda i: (0, i, 0)),
            pl.BlockSpec((_NC, _GBLK, _R), lambda i: (0, i, 0)),
            pl.BlockSpec((_GBLK, _D), lambda i: (i, 0)),
            pl.BlockSpec((_R, _D), lambda i: (0, 0)),
            pl.BlockSpec((_D, _D), lambda i: (0, 0)),
            pl.BlockSpec((_D, _D), lambda i: (0, 0)),
        ],
        out_specs=pl.BlockSpec((_GBLK, _D), lambda i: (i, 0)),
        out_shape=jax.ShapeDtypeStruct((_N, _D), jnp.float32),
    )(s_parts, c_parts, x, rel_emb, W, W_self)


_TBLK = 512


def _mlp_body(ta_ref, tb_ref, w1_ref, b1_ref, w2_ref, b2_ref, oa_ref, ob_ref):
    w1 = w1_ref[...]
    w2 = w2_ref[...]
    b1 = b1_ref[...]
    b2 = b2_ref[...]
    ha = jnp.maximum(jnp.dot(ta_ref[...], w1, precision=lax.Precision.HIGHEST) + b1, 0.0)
    oa_ref[...] = jnp.dot(ha, w2, precision=lax.Precision.HIGHEST) + b2
    hb = jnp.maximum(jnp.dot(tb_ref[...], w1, precision=lax.Precision.HIGHEST) + b1, 0.0)
    ob_ref[...] = jnp.dot(hb, w2, precision=lax.Precision.HIGHEST) + b2


def _text_mlp(ta, tb, W1, b1, W2, b2):
    return pl.pallas_call(
        _mlp_body,
        grid=(_B // _TBLK,),
        in_specs=[
            pl.BlockSpec((_TBLK, _TD), lambda i: (i, 0)),
            pl.BlockSpec((_TBLK, _TD), lambda i: (i, 0)),
            pl.BlockSpec((_TD, 2 * _D), lambda i: (0, 0)),
            pl.BlockSpec((1, 2 * _D), lambda i: (0, 0)),
            pl.BlockSpec((2 * _D, _D), lambda i: (0, 0)),
            pl.BlockSpec((1, _D), lambda i: (0, 0)),
        ],
        out_specs=[
            pl.BlockSpec((_TBLK, _D), lambda i: (i, 0)),
            pl.BlockSpec((_TBLK, _D), lambda i: (i, 0)),
        ],
        out_shape=[
            jax.ShapeDtypeStruct((_B, _D), jnp.float32),
            jax.ShapeDtypeStruct((_B, _D), jnp.float32),
        ],
    )(ta, tb, W1, b1, W2, b2)


def kernel(x, rel_emb, W, W_self, text_features_ab, text_features_bc,
           W1, b1, W2, b2, edge_index, edge_type):
    src = edge_index[0]
    dst = edge_index[1]
    cidx = dst * _R + edge_type
    pad = _EP - _E
    src_p = jnp.concatenate([src, jnp.zeros((pad,), jnp.int32)])
    dst_p = jnp.concatenate([dst, jnp.full((pad,), _NP - 1, jnp.int32)])
    cidx_p = jnp.concatenate([cidx, jnp.full((pad,), _NPR - 1, jnp.int32)])
    src2 = src_p.reshape(_EP // _CH, _CH)
    dst2 = dst_p.reshape(_EP // _CH, _CH)
    cidx2 = cidx_p.reshape(_EP // _CH, _CH)

    s_parts, c_parts = _sc_scatter()(x, src2, dst2, cidx2)
    c_parts = c_parts.reshape(_NC, _NP, _R)

    proj_ab, proj_bc = _text_mlp(text_features_ab, text_features_bc,
                                 W1, b1.reshape(1, -1), W2, b2.reshape(1, -1))
    node = _gnn_finish(s_parts, c_parts, x, rel_emb, W, W_self)
    return (node, proj_ab, proj_bc)
